# baseline (speedup 1.0000x reference)
import functools

import jax
import jax.numpy as jnp
from jax import lax
from jax.experimental import pallas as pl
from jax.experimental.pallas import tpu as pltpu
from jax.experimental.pallas import tpu_sc as plsc

_NUM_CORES = 2
_NUM_SUBCORES = 16
_NUM_WORKERS = _NUM_CORES * _NUM_SUBCORES


def _iota16():
    return jnp.arange(16, dtype=jnp.int32)


def _detile_kernel(v, tab_t, flat_out, in0, in1, out0, out1, in_tail,
                   out_tail, sem_r0, sem_r1, sem_w0, sem_w1, sem_t):
    sb = 512
    nblk = v // sb
    tail = v - nblk * sb
    wid = lax.axis_index("s") * _NUM_CORES + lax.axis_index("c")
    cnt = (nblk - wid + _NUM_WORKERS - 1) // _NUM_WORKERS

    ins = (in0, in1)
    outs = (out0, out1)
    sem_r = (sem_r0, sem_r1)
    sem_w = (sem_w0, sem_w1)
    iota = _iota16()

    def read(k, bf):
        blk = (wid + k * _NUM_WORKERS) * sb
        return pltpu.make_async_copy(
            tab_t.at[:, pl.ds(blk, sb)], ins[bf], sem_r[bf])

    def write(k, bf):
        blk = wid + k * _NUM_WORKERS
        return pltpu.make_async_copy(
            outs[bf], flat_out.at[pl.ds(blk * (sb // 4), sb // 4)], sem_w[bf])

    def transpose_block(src, dst, ncols):
        lo = iota
        hi = iota + 16

        @plsc.parallel_loop(0, ncols // 4, unroll=2)
        def _(qq):
            q0 = qq * 4
            loaded = []
            for j in range(4):
                qcol = jnp.full((16,), q0 + j, dtype=jnp.int32)
                loaded.append((plsc.load_gather(src, [lo, qcol]),
                               plsc.load_gather(src, [hi, qcol])))
            row = q0 // 4
            for j, (v0, v1) in enumerate(loaded):
                dst[row, pl.ds(j * 32, 16)] = v0
                dst[row, pl.ds(j * 32 + 16, 16)] = v1

    read(0, 0).start()
    read(1, 1).start()

    def body(i, c):
        for bf in range(2):
            k = 2 * i + bf

            @pl.when(k < cnt)
            def _():
                read(k, bf).wait()

                @pl.when(k >= 2)
                def _():
                    write(k - 2, bf).wait()

                transpose_block(ins[bf], outs[bf], sb)
                write(k, bf).start()

                @pl.when(k + 2 < cnt)
                def _():
                    read(k + 2, bf).start()
        return c

    lax.fori_loop(0, (cnt + 1) // 2, body, 0)

    write(0, 0).wait()
    write(0, 1).wait()

    if tail:
        @pl.when(wid == _NUM_WORKERS - 1)
        def _():
            pltpu.sync_copy(tab_t.at[:, pl.ds(nblk * sb, tail)], in_tail)
            transpose_block(in_tail, out_tail, tail)
            pltpu.make_async_copy(
                out_tail, flat_out.at[pl.ds(nblk * (sb // 4), tail // 4)],
                sem_t).start()
            pltpu.make_async_copy(
                out_tail, flat_out.at[pl.ds(nblk * (sb // 4), tail // 4)],
                sem_t).wait()


def _gather_kernel(b, h, idx_hbm, table_hbm, out5_hbm, idx_v, it0, it1,
                   rows0, rows1, tile0, tile1, sem_g0, sem_g1, sem_w0, sem_w1):
    rows_per_w = b // _NUM_WORKERS
    wid = lax.axis_index("s") * _NUM_CORES + lax.axis_index("c")

    idx_t = (it0, it1)
    rows = (rows0, rows1)
    tile = (tile0, tile1)
    sem_g = (sem_g0, sem_g1)
    sem_w = (sem_w0, sem_w1)
    iota = _iota16()
    d8_0 = iota // 8
    d8_1 = d8_0 + 2
    d8r = iota % 8

    pltpu.sync_copy(idx_hbm.at[pl.ds(wid * rows_per_w * h, rows_per_w * h)],
                    idx_v)

    def build_idx(t, bf):
        dst = idx_t[bf]
        for g in range(rows_per_w // 16):
            pos = (iota + g * 16) * h + t
            dst[pl.ds(g * 16, 16)] = plsc.load_gather(idx_v, [pos])

    def gather(bf):
        return pltpu.make_async_copy(
            table_hbm.at[idx_t[bf]], rows[bf], sem_g[bf])

    def write(t, bf):
        return pltpu.make_async_copy(
            tile[bf], out5_hbm.at[t, :, pl.ds(wid * 4, 4)], sem_w[bf])

    def transpose_rows(bf):
        src = rows[bf]
        dst = tile[bf]
        dcols = [jnp.full((16,), d, dtype=jnp.int32) for d in range(32)]

        @plsc.parallel_loop(0, rows_per_w // 16, unroll=2)
        def _(g):
            bidx = iota + g * 16
            b128 = g // 8
            coff = (g % 8) * 16
            for d0 in range(0, 32, 8):
                vs = [plsc.load_gather(src, [bidx, dcols[d0 + j]])
                      for j in range(8)]
                for j in range(8):
                    d = d0 + j
                    dst[d // 8, b128, d % 8, pl.ds(coff, 16)] = vs[j]

    build_idx(0, 0)
    gather(0).start()
    build_idx(1, 1)
    gather(1).start()

    assert h % 2 == 0

    def body(i, c):
        for bf in range(2):
            t = 2 * i + bf
            gather(bf).wait()

            @pl.when(t >= 2)
            def _():
                write(t - 2, bf).wait()

            transpose_rows(bf)
            write(t, bf).start()

            @pl.when(t + 2 < h)
            def _():
                build_idx(t + 2, bf)
                gather(bf).start()
        return c

    lax.fori_loop(0, h // 2, body, 0)
    write(0, 0).wait()
    write(0, 1).wait()


def kernel(indices, embedding):
    b, h = indices.shape
    v, d = embedding.shape
    n = b * h
    assert d == 32 and b % (128 * _NUM_WORKERS) == 0 and v % 4 == 0

    flat_idx = indices.reshape(n).astype(jnp.int32)
    mesh = plsc.VectorSubcoreMesh(core_axis_name="c", subcore_axis_name="s")

    tail = v - (v // 512) * 512
    detile = functools.partial(
        pl.kernel,
        mesh=mesh,
        out_type=jax.ShapeDtypeStruct((v // 4, 128), jnp.float32),
        scratch_types=[
            pltpu.VMEM((32, 512), jnp.float32),
            pltpu.VMEM((32, 512), jnp.float32),
            pltpu.VMEM((128, 128), jnp.float32),
            pltpu.VMEM((128, 128), jnp.float32),
            pltpu.VMEM((32, max(tail, 1)), jnp.float32),
            pltpu.VMEM((max(tail // 4, 1), 128), jnp.float32),
            pltpu.SemaphoreType.DMA,
            pltpu.SemaphoreType.DMA,
            pltpu.SemaphoreType.DMA,
            pltpu.SemaphoreType.DMA,
            pltpu.SemaphoreType.DMA,
        ],
        compiler_params=pltpu.CompilerParams(use_tc_tiling_on_sc=True, needs_layout_passes=False),
    )(functools.partial(_detile_kernel, v))

    table_flat = detile(embedding.T).reshape(v, d)

    rows_per_w = b // _NUM_WORKERS
    gather_run = functools.partial(
        pl.kernel,
        mesh=mesh,
        out_type=jax.ShapeDtypeStruct((h, 4, b // 128, 8, 128), jnp.float32),
        scratch_types=[
            pltpu.VMEM((rows_per_w * h,), jnp.int32),
            pltpu.VMEM((rows_per_w,), jnp.int32),
            pltpu.VMEM((rows_per_w,), jnp.int32),
            pltpu.VMEM((rows_per_w, d), jnp.float32),
            pltpu.VMEM((rows_per_w, d), jnp.float32),
            pltpu.VMEM((4, 4, 8, 128), jnp.float32),
            pltpu.VMEM((4, 4, 8, 128), jnp.float32),
            pltpu.SemaphoreType.DMA,
            pltpu.SemaphoreType.DMA,
            pltpu.SemaphoreType.DMA,
            pltpu.SemaphoreType.DMA,
        ],
        compiler_params=pltpu.CompilerParams(use_tc_tiling_on_sc=False, needs_layout_passes=False),
    )(functools.partial(_gather_kernel, b, h))

    out5 = gather_run(flat_idx, table_flat)
    return out5.transpose(2, 4, 0, 1, 3).reshape(b, h, d)


# speedup vs baseline: 1.0674x; 1.0674x over previous
import functools

import jax
import jax.numpy as jnp
from jax import lax
from jax.experimental import pallas as pl
from jax.experimental.pallas import tpu as pltpu
from jax.experimental.pallas import tpu_sc as plsc

_NUM_CORES = 2
_NUM_SUBCORES = 16
_NUM_WORKERS = _NUM_CORES * _NUM_SUBCORES


def _iota16():
    return jnp.arange(16, dtype=jnp.int32)


def _detile_kernel(v, tab_t, flat_out, in0, in1, out0, out1, in_tail,
                   out_tail, sem_r0, sem_r1, sem_w0, sem_w1, sem_t):
    sb = 128
    nblk = v // sb
    tail = v - nblk * sb
    wid = lax.axis_index("s") * _NUM_CORES + lax.axis_index("c")
    cnt = (nblk - wid + _NUM_WORKERS - 1) // _NUM_WORKERS

    ins = (in0, in1)
    outs = (out0, out1)
    sem_r = (sem_r0, sem_r1)
    sem_w = (sem_w0, sem_w1)
    iota = _iota16()

    def read(k, bf):
        blk = (wid + k * _NUM_WORKERS) * sb
        return pltpu.make_async_copy(
            tab_t.at[:, pl.ds(blk, sb)], ins[bf], sem_r[bf])

    def write(k, bf):
        blk = wid + k * _NUM_WORKERS
        return pltpu.make_async_copy(
            outs[bf], flat_out.at[pl.ds(blk * (sb // 4), sb // 4)], sem_w[bf])

    def transpose_block(src, dst, ncols):
        lo = iota
        hi = iota + 16

        @plsc.parallel_loop(0, ncols // 4, unroll=2)
        def _(qq):
            q0 = qq * 4
            loaded = []
            for j in range(4):
                qcol = jnp.full((16,), q0 + j, dtype=jnp.int32)
                loaded.append((plsc.load_gather(src, [lo, qcol]),
                               plsc.load_gather(src, [hi, qcol])))
            row = q0 // 4
            for j, (v0, v1) in enumerate(loaded):
                dst[row, pl.ds(j * 32, 16)] = v0
                dst[row, pl.ds(j * 32 + 16, 16)] = v1

    read(0, 0).start()
    read(1, 1).start()

    def body(i, c):
        for bf in range(2):
            k = 2 * i + bf

            @pl.when(k < cnt)
            def _():
                read(k, bf).wait()

                @pl.when(k >= 2)
                def _():
                    write(k - 2, bf).wait()

                transpose_block(ins[bf], outs[bf], sb)
                write(k, bf).start()

                @pl.when(k + 2 < cnt)
                def _():
                    read(k + 2, bf).start()
        return c

    lax.fori_loop(0, (cnt + 1) // 2, body, 0)

    write(0, 0).wait()
    write(0, 1).wait()

    if tail:
        @pl.when(wid == _NUM_WORKERS - 1)
        def _():
            pltpu.sync_copy(tab_t.at[:, pl.ds(nblk * sb, tail)], in_tail)
            transpose_block(in_tail, out_tail, tail)
            pltpu.make_async_copy(
                out_tail, flat_out.at[pl.ds(nblk * (sb // 4), tail // 4)],
                sem_t).start()
            pltpu.make_async_copy(
                out_tail, flat_out.at[pl.ds(nblk * (sb // 4), tail // 4)],
                sem_t).wait()


def _gather_kernel(b, h, idx_hbm, table_hbm, out5_hbm, idx_v, it0, it1,
                   rows0, rows1, tile0, tile1, sem_g0, sem_g1, sem_w0, sem_w1):
    rows_per_w = b // _NUM_WORKERS
    wid = lax.axis_index("s") * _NUM_CORES + lax.axis_index("c")

    idx_t = (it0, it1)
    rows = (rows0, rows1)
    tile = (tile0, tile1)
    sem_g = (sem_g0, sem_g1)
    sem_w = (sem_w0, sem_w1)
    iota = _iota16()
    d8_0 = iota // 8
    d8_1 = d8_0 + 2
    d8r = iota % 8

    pltpu.sync_copy(idx_hbm.at[pl.ds(wid * rows_per_w * h, rows_per_w * h)],
                    idx_v)

    def build_idx(t, bf):
        dst = idx_t[bf]
        for g in range(rows_per_w // 16):
            pos = (iota + g * 16) * h + t
            dst[pl.ds(g * 16, 16)] = plsc.load_gather(idx_v, [pos])

    def gather(bf):
        return pltpu.make_async_copy(
            table_hbm.at[idx_t[bf]], rows[bf], sem_g[bf])

    def write(t, bf):
        return pltpu.make_async_copy(
            tile[bf], out5_hbm.at[t, :, pl.ds(wid * 4, 4)], sem_w[bf])

    def transpose_rows(bf):
        src = rows[bf]
        dst = tile[bf]
        dcols = [jnp.full((16,), d, dtype=jnp.int32) for d in range(32)]

        @plsc.parallel_loop(0, rows_per_w // 16, unroll=2)
        def _(g):
            bidx = iota + g * 16
            b128 = g // 8
            coff = (g % 8) * 16
            for d0 in range(0, 32, 8):
                vs = [plsc.load_gather(src, [bidx, dcols[d0 + j]])
                      for j in range(8)]
                for j in range(8):
                    d = d0 + j
                    dst[d // 8, b128, d % 8, pl.ds(coff, 16)] = vs[j]

    build_idx(0, 0)
    gather(0).start()
    build_idx(1, 1)
    gather(1).start()

    assert h % 2 == 0

    def body(i, c):
        for bf in range(2):
            t = 2 * i + bf
            gather(bf).wait()

            @pl.when(t >= 2)
            def _():
                write(t - 2, bf).wait()

            transpose_rows(bf)
            write(t, bf).start()

            @pl.when(t + 2 < h)
            def _():
                build_idx(t + 2, bf)
                gather(bf).start()
        return c

    lax.fori_loop(0, h // 2, body, 0)
    write(0, 0).wait()
    write(0, 1).wait()


def kernel(indices, embedding):
    b, h = indices.shape
    v, d = embedding.shape
    n = b * h
    assert d == 32 and b % (128 * _NUM_WORKERS) == 0 and v % 4 == 0

    flat_idx = indices.reshape(n).astype(jnp.int32)
    mesh = plsc.VectorSubcoreMesh(core_axis_name="c", subcore_axis_name="s")

    tail = v - (v // 128) * 128
    detile = functools.partial(
        pl.kernel,
        mesh=mesh,
        out_type=jax.ShapeDtypeStruct((v // 4, 128), jnp.float32),
        scratch_types=[
            pltpu.VMEM((32, 128), jnp.float32),
            pltpu.VMEM((32, 128), jnp.float32),
            pltpu.VMEM((32, 128), jnp.float32),
            pltpu.VMEM((32, 128), jnp.float32),
            pltpu.VMEM((32, max(tail, 1)), jnp.float32),
            pltpu.VMEM((max(tail // 4, 1), 128), jnp.float32),
            pltpu.SemaphoreType.DMA,
            pltpu.SemaphoreType.DMA,
            pltpu.SemaphoreType.DMA,
            pltpu.SemaphoreType.DMA,
            pltpu.SemaphoreType.DMA,
        ],
        compiler_params=pltpu.CompilerParams(use_tc_tiling_on_sc=True, needs_layout_passes=False),
    )(functools.partial(_detile_kernel, v))

    table_flat = detile(embedding.T).reshape(v, d)

    rows_per_w = b // _NUM_WORKERS
    gather_run = functools.partial(
        pl.kernel,
        mesh=mesh,
        out_type=jax.ShapeDtypeStruct((h, 4, b // 128, 8, 128), jnp.float32),
        scratch_types=[
            pltpu.VMEM((rows_per_w * h,), jnp.int32),
            pltpu.VMEM((rows_per_w,), jnp.int32),
            pltpu.VMEM((rows_per_w,), jnp.int32),
            pltpu.VMEM((rows_per_w, d), jnp.float32),
            pltpu.VMEM((rows_per_w, d), jnp.float32),
            pltpu.VMEM((4, 4, 8, 128), jnp.float32),
            pltpu.VMEM((4, 4, 8, 128), jnp.float32),
            pltpu.SemaphoreType.DMA,
            pltpu.SemaphoreType.DMA,
            pltpu.SemaphoreType.DMA,
            pltpu.SemaphoreType.DMA,
        ],
        compiler_params=pltpu.CompilerParams(use_tc_tiling_on_sc=False, needs_layout_passes=False),
    )(functools.partial(_gather_kernel, b, h))

    out5 = gather_run(flat_idx, table_flat)
    return out5.transpose(2, 4, 0, 1, 3).reshape(b, h, d)
